# baseline (speedup 1.0000x reference)
import functools

import jax
import jax.numpy as jnp
from jax import lax
from jax.experimental import pallas as pl
from jax.experimental.pallas import tpu as pltpu
from jax.experimental.pallas import tpu_sc as plsc

_K = 7
_D = 16
_DO = _D - _K + 1
_H = 64
_HO = _H - _K + 1
_C = 3

_NWORK = 32
_NFLAT = _DO * _H * _H
_CHUNK = _NFLAT // _NWORK


def _dist_kernel(x_ref, y_ref, d_ref):
    xs = x_ref[...]
    ys = y_ref[...]

    Dm = []
    for delta in range(-(_DO - 1), _DO):
        a_lo = max(0, -delta)
        a_hi = min(_D - 1, _D - 1 - delta)
        na = a_hi - a_lo + 1
        diff = xs[:, a_lo:a_hi + 1] - ys[:, a_lo + delta:a_hi + 1 + delta]
        z = jnp.sum(diff * diff, axis=0)
        zh = z[:, 0:_HO, :]
        for t in range(1, _K):
            zh = zh + z[:, t:t + _HO, :]
        zw = zh[:, :, 0:_HO]
        for t in range(1, _K):
            zw = zw + zh[:, :, t:t + _HO]
        ndi = na - _K + 1
        dsum = zw[0:ndi]
        for t in range(1, _K):
            dsum = dsum + zw[t:t + ndi]
        dsum = jnp.pad(dsum, ((a_lo, _DO - ndi - a_lo), (0, 0), (0, 0)))
        Dm.append(dsum)

    rows = []
    for dj in range(_DO):
        planes = [jnp.pad(Dm[dj - di + _DO - 1][di:di + 1],
                          ((0, 0), (0, _H - _HO), (0, _H - _HO)))
                  for di in range(_DO)]
        rows.append(jnp.concatenate(planes, axis=0)[None])
    d_ref[...] = jnp.concatenate(rows, axis=0)


def _sc_argmin(d_hbm, nns_hbm, d_v, nn_v):
    wid = lax.axis_index("s") * 2 + lax.axis_index("c")
    base = wid * _CHUNK
    for dj in range(_DO):
        pltpu.sync_copy(d_hbm.at[dj, pl.ds(base, _CHUNK)], d_v.at[dj])

    def body(i, carry):
        off = i * 16
        mv = d_v[0, pl.ds(off, 16)]
        mi = jnp.zeros((16,), jnp.int32)
        for dj in range(1, _DO):
            v = d_v[dj, pl.ds(off, 16)]
            p = v < mv
            mi = jnp.where(p, jnp.int32(dj), mi)
            mv = jnp.where(p, v, mv)
        nn_v[pl.ds(off, 16)] = mi
        return carry

    lax.fori_loop(0, _CHUNK // 16, body, 0)
    pltpu.sync_copy(nn_v, nns_hbm.at[pl.ds(base, _CHUNK)])


def _fold_kernel(nns_ref, x_ref, y_ref, out_ref):
    xs = x_ref[...]
    ys = y_ref[...]
    minidx = nns_ref[...]

    eye = lax.broadcasted_iota(jnp.int32, (1, _DO, 1, 1), 1)
    I = (minidx[:, None] == eye).astype(jnp.float32)
    Ip = jnp.pad(I, ((0, 0), (0, 0), (6, 6), (6, 6)))
    Jh = Ip[:, :, 0:_H, 6:6 + _HO]
    for t in range(1, _K):
        Jh = Jh + Ip[:, :, t:t + _H, 6:6 + _HO]
    Jp = jnp.pad(Jh, ((0, 0), (0, 0), (0, 0), (6, 6)))
    J = Jp[:, :, :, 0:_H]
    for t in range(1, _K):
        J = J + Jp[:, :, :, t:t + _H]

    accs = []
    for di in range(_DO):
        acc = J[di, 0][None, None] * ys[:, 0:_K]
        for e0 in range(1, _DO):
            acc = acc + J[di, e0][None, None] * ys[:, e0:e0 + _K]
        accs.append(acc)
    slabs = []
    for dv in range(_D):
        terms = [accs[di][:, dv - di]
                 for di in range(max(0, dv - _K + 1), min(_DO - 1, dv) + 1)]
        s = terms[0]
        for t in terms[1:]:
            s = s + t
        slabs.append(s[:, None])
    folded = jnp.concatenate(slabs, axis=1)

    dvi = lax.broadcasted_iota(jnp.int32, (_D, _H, _H), 0).astype(jnp.float32)
    hvi = lax.broadcasted_iota(jnp.int32, (_D, _H, _H), 1).astype(jnp.float32)
    wvi = lax.broadcasted_iota(jnp.int32, (_D, _H, _H), 2).astype(jnp.float32)
    covD = jnp.minimum(jnp.minimum(dvi + 1.0, float(_K)), float(_D) - dvi)
    covH = (jnp.minimum(hvi, float(_HO - 1))
            - jnp.maximum(hvi - float(_K - 1), 0.0) + 1.0)
    covW = (jnp.minimum(wvi, float(_HO - 1))
            - jnp.maximum(wvi - float(_K - 1), 0.0) + 1.0)
    wgt = covD * covH * covW

    total = jnp.sum(jnp.abs(xs * wgt[None] - folded))
    out_ref[0, 0] = total / float(_C * _D * _H * _H)


def kernel(x, y):
    xs = x.reshape(_C, _D, _H, _H)
    ys = y.reshape(_C, _D, _H, _H)

    dstack = pl.pallas_call(
        _dist_kernel,
        out_shape=jax.ShapeDtypeStruct((_DO, _DO, _H, _H), jnp.float32),
    )(xs, ys)

    mesh = plsc.VectorSubcoreMesh(core_axis_name="c", subcore_axis_name="s")
    sc_argmin = functools.partial(
        pl.kernel, mesh=mesh,
        out_type=jax.ShapeDtypeStruct((_NFLAT,), jnp.int32),
        scratch_types=[
            pltpu.VMEM((_DO, _CHUNK), jnp.float32),
            pltpu.VMEM((_CHUNK,), jnp.int32),
        ],
    )(_sc_argmin)
    nns_flat = sc_argmin(dstack.reshape(_DO, _NFLAT))
    nns = nns_flat.reshape(_DO, _H, _H)[:, :_HO, :_HO]

    out = pl.pallas_call(
        _fold_kernel,
        out_shape=jax.ShapeDtypeStruct((1, 1), jnp.float32),
        out_specs=pl.BlockSpec(memory_space=pltpu.SMEM),
    )(nns, xs, ys)
    return out.reshape(())


# speedup vs baseline: 1.3032x; 1.3032x over previous
import functools

import jax
import jax.numpy as jnp
from jax import lax
from jax.experimental import pallas as pl
from jax.experimental.pallas import tpu as pltpu
from jax.experimental.pallas import tpu_sc as plsc

_K = 7
_D = 16
_DO = _D - _K + 1
_H = 64
_HO = _H - _K + 1
_C = 3
_P = _H * _H

_NWORK = 32
_NFLAT = _DO * _P
_CHUNK = _NFLAT // _NWORK


def _shl(v, t):
    return jnp.concatenate([v[..., t:], v[..., :t]], axis=-1)


def _shr(v, t):
    return jnp.concatenate([v[..., _P - t:], v[..., :_P - t]], axis=-1)


def _box7(v, sh, step):
    p1 = v + sh(v, step)
    p2 = p1 + sh(p1, 2 * step)
    return p2 + sh(p1, 4 * step) + sh(v, 6 * step)


def _dist_kernel(x_ref, y_ref, d_ref):
    xs = x_ref[...]
    ys = y_ref[...]

    big = jnp.full((1, _P), 3e38, dtype=jnp.float32)
    Dm = []
    for delta in range(-(_DO - 1), _DO):
        a_lo = max(0, -delta)
        a_hi = min(_D - 1, _D - 1 - delta)
        na = a_hi - a_lo + 1
        diff = xs[:, a_lo:a_hi + 1] - ys[:, a_lo + delta:a_hi + 1 + delta]
        z = jnp.sum(diff * diff, axis=0)
        zw = _box7(z, _shl, 1)
        zh = _box7(zw, _shl, _H)
        ndi = na - _K + 1
        dsum = zh[0:ndi]
        for t in range(1, _K):
            dsum = dsum + zh[t:t + ndi]
        pads = ([big] * a_lo + [dsum] + [big] * (_DO - ndi - a_lo))
        Dm.append(jnp.concatenate(pads, axis=0) if len(pads) > 1 else dsum)

    rows = []
    for dj in range(_DO):
        planes = [Dm[dj - di + _DO - 1][di:di + 1] for di in range(_DO)]
        rows.append(jnp.concatenate(planes, axis=0)[None])
    d_ref[...] = jnp.concatenate(rows, axis=0)


def _sc_argmin(d_hbm, nns_hbm, d_v, nn_v):
    wid = lax.axis_index("s") * 2 + lax.axis_index("c")
    base = wid * _CHUNK
    for dj in range(_DO):
        pltpu.sync_copy(d_hbm.at[dj, pl.ds(base, _CHUNK)], d_v.at[dj])

    def body(i, carry):
        off = i * 16
        mv = d_v[0, pl.ds(off, 16)]
        mi = jnp.zeros((16,), jnp.int32)
        for dj in range(1, _DO):
            v = d_v[dj, pl.ds(off, 16)]
            p = v < mv
            mi = jnp.where(p, jnp.int32(dj), mi)
            mv = jnp.where(p, v, mv)
        nn_v[pl.ds(off, 16)] = mi
        return carry

    lax.fori_loop(0, _CHUNK // 16, body, 0)
    pltpu.sync_copy(nn_v, nns_hbm.at[pl.ds(base, _CHUNK)])


def _fold_kernel(nns_ref, x_ref, y_ref, out_ref):
    xs = x_ref[...]
    ys = y_ref[...]
    minidx = nns_ref[...]

    ii = lax.broadcasted_iota(jnp.int32, (1, _P), 1)
    valid = jnp.logical_and(ii // _H < _HO, ii % _H < _HO)

    J = []
    for di in range(_DO):
        row = []
        for e0 in range(_DO):
            ind = jnp.where(
                jnp.logical_and(minidx[di:di + 1] == e0, valid), 1.0, 0.0)
            row.append(ind)
        m = jnp.concatenate(row, axis=0)
        m = _box7(m, _shr, 1)
        m = _box7(m, _shr, _H)
        J.append(m)

    accs = []
    for di in range(_DO):
        acc = J[di][0][None, None] * ys[:, 0:_K]
        for e0 in range(1, _DO):
            acc = acc + J[di][e0][None, None] * ys[:, e0:e0 + _K]
        accs.append(acc)
    slabs = []
    for dv in range(_D):
        terms = [accs[di][:, dv - di]
                 for di in range(max(0, dv - _K + 1), min(_DO - 1, dv) + 1)]
        s = terms[0]
        for t in terms[1:]:
            s = s + t
        slabs.append(s[:, None])
    folded = jnp.concatenate(slabs, axis=1)

    di2 = lax.broadcasted_iota(jnp.int32, (_D, _P), 0).astype(jnp.float32)
    pi = lax.broadcasted_iota(jnp.int32, (_D, _P), 1)
    hvi = (pi // _H).astype(jnp.float32)
    wvi = (pi % _H).astype(jnp.float32)
    covD = jnp.minimum(jnp.minimum(di2 + 1.0, float(_K)), float(_D) - di2)
    covH = (jnp.minimum(hvi, float(_HO - 1))
            - jnp.maximum(hvi - float(_K - 1), 0.0) + 1.0)
    covW = (jnp.minimum(wvi, float(_HO - 1))
            - jnp.maximum(wvi - float(_K - 1), 0.0) + 1.0)
    wgt = covD * covH * covW

    total = jnp.sum(jnp.abs(xs * wgt[None] - folded))
    out_ref[0, 0] = total / float(_C * _D * _P)


def kernel(x, y):
    xs = x.reshape(_C, _D, _P)
    ys = y.reshape(_C, _D, _P)

    dstack = pl.pallas_call(
        _dist_kernel,
        out_shape=jax.ShapeDtypeStruct((_DO, _DO, _P), jnp.float32),
    )(xs, ys)

    mesh = plsc.VectorSubcoreMesh(core_axis_name="c", subcore_axis_name="s")
    sc_argmin = functools.partial(
        pl.kernel, mesh=mesh,
        out_type=jax.ShapeDtypeStruct((_NFLAT,), jnp.int32),
        scratch_types=[
            pltpu.VMEM((_DO, _CHUNK), jnp.float32),
            pltpu.VMEM((_CHUNK,), jnp.int32),
        ],
    )(_sc_argmin)
    nns_flat = sc_argmin(dstack.reshape(_DO, _NFLAT))

    out = pl.pallas_call(
        _fold_kernel,
        out_shape=jax.ShapeDtypeStruct((1, 1), jnp.float32),
        out_specs=pl.BlockSpec(memory_space=pltpu.SMEM),
    )(nns_flat.reshape(_DO, _P), xs, ys)
    return out.reshape(())


# speedup vs baseline: 1.4521x; 1.1143x over previous
import functools

import jax
import jax.numpy as jnp
from jax import lax
from jax.experimental import pallas as pl
from jax.experimental.pallas import tpu as pltpu
from jax.experimental.pallas import tpu_sc as plsc

_K = 7
_D = 16
_DO = _D - _K + 1
_H = 64
_HO = _H - _K + 1
_C = 3
_P = _H * _H

_NWORK = 32
_NFLAT = _DO * _P
_CHUNK = _NFLAT // _NWORK


def _shl(v, t):
    return jnp.concatenate([v[..., t:], v[..., :t]], axis=-1)


def _shr(v, t):
    return jnp.concatenate([v[..., _P - t:], v[..., :_P - t]], axis=-1)


def _box7(v, sh, step):
    p1 = v + sh(v, step)
    p2 = p1 + sh(p1, 2 * step)
    return p2 + sh(p1, 4 * step) + sh(v, 6 * step)


def _dist_kernel(x_ref, y_ref, d_ref):
    xs = x_ref[...]
    ys = y_ref[...]

    big = jnp.full((1, _P), 3e38, dtype=jnp.float32)
    Dm = []
    for delta in range(-(_DO - 1), _DO):
        a_lo = max(0, -delta)
        a_hi = min(_D - 1, _D - 1 - delta)
        na = a_hi - a_lo + 1
        diff = xs[:, a_lo:a_hi + 1] - ys[:, a_lo + delta:a_hi + 1 + delta]
        z = jnp.sum(diff * diff, axis=0)
        zw = _box7(z, _shl, 1)
        zh = _box7(zw, _shl, _H)
        ndi = na - _K + 1
        dsum = zh[0:ndi]
        for t in range(1, _K):
            dsum = dsum + zh[t:t + ndi]
        pads = ([big] * a_lo + [dsum] + [big] * (_DO - ndi - a_lo))
        Dm.append(jnp.concatenate(pads, axis=0) if len(pads) > 1 else dsum)

    rows = []
    for dj in range(_DO):
        planes = [Dm[dj - di + _DO - 1][di:di + 1] for di in range(_DO)]
        rows.append(jnp.concatenate(planes, axis=1))
    d_ref[...] = jnp.concatenate(rows, axis=0)


def _sc_argmin(d_hbm, nns_hbm, d_v, nn_v):
    wid = lax.axis_index("s") * 2 + lax.axis_index("c")
    base = wid * _CHUNK
    for dj in range(_DO):
        pltpu.sync_copy(d_hbm.at[dj, pl.ds(base, _CHUNK)], d_v.at[dj])

    def body(i, carry):
        off = i * 16
        mv = d_v[0, pl.ds(off, 16)]
        mi = jnp.zeros((16,), jnp.int32)
        for dj in range(1, _DO):
            v = d_v[dj, pl.ds(off, 16)]
            p = v < mv
            mi = jnp.where(p, jnp.int32(dj), mi)
            mv = jnp.where(p, v, mv)
        nn_v[pl.ds(off, 16)] = mi
        return carry

    lax.fori_loop(0, _CHUNK // 16, body, 0, unroll=4)
    pltpu.sync_copy(nn_v, nns_hbm.at[pl.ds(base, _CHUNK)])


def _fold_kernel(nns_ref, x_ref, y_ref, out_ref):
    xs = x_ref[...]
    ys = y_ref[...]

    ii = lax.broadcasted_iota(jnp.int32, (1, _P), 1)
    valid = jnp.logical_and(ii // _H < _HO, ii % _H < _HO)

    J = []
    for di in range(_DO):
        midx = nns_ref[pl.ds(di * _P, _P)].reshape(1, _P)
        row = []
        for e0 in range(_DO):
            ind = jnp.where(
                jnp.logical_and(midx == e0, valid), 1.0, 0.0)
            row.append(ind)
        m = jnp.concatenate(row, axis=0)
        m = _box7(m, _shr, 1)
        m = _box7(m, _shr, _H)
        J.append(m)

    accs = []
    for di in range(_DO):
        acc = J[di][0][None, None] * ys[:, 0:_K]
        for e0 in range(1, _DO):
            acc = acc + J[di][e0][None, None] * ys[:, e0:e0 + _K]
        accs.append(acc)
    slabs = []
    for dv in range(_D):
        terms = [accs[di][:, dv - di]
                 for di in range(max(0, dv - _K + 1), min(_DO - 1, dv) + 1)]
        s = terms[0]
        for t in terms[1:]:
            s = s + t
        slabs.append(s[:, None])
    folded = jnp.concatenate(slabs, axis=1)

    di2 = lax.broadcasted_iota(jnp.int32, (_D, _P), 0).astype(jnp.float32)
    pi = lax.broadcasted_iota(jnp.int32, (_D, _P), 1)
    hvi = (pi // _H).astype(jnp.float32)
    wvi = (pi % _H).astype(jnp.float32)
    covD = jnp.minimum(jnp.minimum(di2 + 1.0, float(_K)), float(_D) - di2)
    covH = (jnp.minimum(hvi, float(_HO - 1))
            - jnp.maximum(hvi - float(_K - 1), 0.0) + 1.0)
    covW = (jnp.minimum(wvi, float(_HO - 1))
            - jnp.maximum(wvi - float(_K - 1), 0.0) + 1.0)
    wgt = covD * covH * covW

    total = jnp.sum(jnp.abs(xs * wgt[None] - folded))
    out_ref[0, 0] = total / float(_C * _D * _P)


def kernel(x, y):
    xs = x.reshape(_C, _D, _P)
    ys = y.reshape(_C, _D, _P)

    dstack = pl.pallas_call(
        _dist_kernel,
        out_shape=jax.ShapeDtypeStruct((_DO, _NFLAT), jnp.float32),
    )(xs, ys)

    mesh = plsc.VectorSubcoreMesh(core_axis_name="c", subcore_axis_name="s")
    sc_argmin = functools.partial(
        pl.kernel, mesh=mesh,
        out_type=jax.ShapeDtypeStruct((_NFLAT,), jnp.int32),
        scratch_types=[
            pltpu.VMEM((_DO, _CHUNK), jnp.float32),
            pltpu.VMEM((_CHUNK,), jnp.int32),
        ],
    )(_sc_argmin)
    nns_flat = sc_argmin(dstack)

    out = pl.pallas_call(
        _fold_kernel,
        out_shape=jax.ShapeDtypeStruct((1, 1), jnp.float32),
        out_specs=pl.BlockSpec(memory_space=pltpu.SMEM),
    )(nns_flat, xs, ys)
    return out.reshape(())


# speedup vs baseline: 1.6875x; 1.1621x over previous
import functools

import jax
import jax.numpy as jnp
from jax import lax
from jax.experimental import pallas as pl
from jax.experimental.pallas import tpu as pltpu
from jax.experimental.pallas import tpu_sc as plsc

_K = 7
_D = 16
_DO = _D - _K + 1
_H = 64
_HO = _H - _K + 1
_C = 3
_P = _H * _H

_NWORK = 32
_NFLAT = _DO * _P
_CHUNK = _NFLAT // _NWORK


def _shl(v, t):
    return jnp.concatenate([v[..., t:], v[..., :t]], axis=-1)


def _shr(v, t):
    return jnp.concatenate([v[..., _P - t:], v[..., :_P - t]], axis=-1)


def _box7(v, sh, step):
    p1 = v + sh(v, step)
    p2 = p1 + sh(p1, 2 * step)
    return p2 + sh(p1, 4 * step) + sh(v, 6 * step)


def _dist_kernel(x_ref, y_ref, d_ref):
    xs = x_ref[...].reshape(_C, _D, _P)
    ys = y_ref[...].reshape(_C, _D, _P)

    big = jnp.full((1, _P), 3e38, dtype=jnp.float32)
    Dm = []
    for delta in range(-(_DO - 1), _DO):
        a_lo = max(0, -delta)
        a_hi = min(_D - 1, _D - 1 - delta)
        na = a_hi - a_lo + 1
        diff = xs[:, a_lo:a_hi + 1] - ys[:, a_lo + delta:a_hi + 1 + delta]
        z = jnp.sum(diff * diff, axis=0)
        zw = _box7(z, _shl, 1)
        zh = _box7(zw, _shl, _H)
        ndi = na - _K + 1
        dsum = zh[0:ndi]
        for t in range(1, _K):
            dsum = dsum + zh[t:t + ndi]
        pads = ([big] * a_lo + [dsum] + [big] * (_DO - ndi - a_lo))
        Dm.append(jnp.concatenate(pads, axis=0) if len(pads) > 1 else dsum)

    rows = []
    for dj in range(_DO):
        planes = [Dm[dj - di + _DO - 1][di:di + 1] for di in range(_DO)]
        rows.append(jnp.concatenate(planes, axis=1))
    d_ref[...] = jnp.concatenate(rows, axis=0)


def _sc_argmin(d_hbm, nns_hbm, d_v, nn_v, sem):
    wid = lax.axis_index("s") * 2 + lax.axis_index("c")
    base = wid * _CHUNK
    copies = [pltpu.make_async_copy(d_hbm.at[dj, pl.ds(base, _CHUNK)],
                                    d_v.at[dj], sem)
              for dj in range(_DO)]
    for c in copies:
        c.start()
    for c in copies:
        c.wait()

    def body(i, carry):
        off = i * 16
        mv = d_v[0, pl.ds(off, 16)]
        mi = jnp.zeros((16,), jnp.int32)
        for dj in range(1, _DO):
            v = d_v[dj, pl.ds(off, 16)]
            p = v < mv
            mi = jnp.where(p, jnp.int32(dj), mi)
            mv = jnp.where(p, v, mv)
        nn_v[pl.ds(off, 16)] = mi
        return carry

    lax.fori_loop(0, _CHUNK // 16, body, 0, unroll=4)
    pltpu.sync_copy(nn_v, nns_hbm.at[pl.ds(base, _CHUNK)])


def _fold_kernel(nns_ref, x_ref, y_ref, out_ref):
    xs = x_ref[...].reshape(_C, _D, _P)
    ys = y_ref[...].reshape(_C, _D, _P)

    ii = lax.broadcasted_iota(jnp.int32, (1, _P), 1)
    valid = jnp.logical_and(ii // _H < _HO, ii % _H < _HO)

    J = []
    for di in range(_DO):
        midx = nns_ref[pl.ds(di * _P, _P)].reshape(1, _P)
        row = []
        for e0 in range(_DO):
            ind = jnp.where(
                jnp.logical_and(midx == e0, valid), 1.0, 0.0)
            row.append(ind)
        m = jnp.concatenate(row, axis=0)
        m = _box7(m, _shr, 1)
        m = _box7(m, _shr, _H)
        J.append(m)

    accs = []
    for di in range(_DO):
        acc = J[di][0][None, None] * ys[:, 0:_K]
        for e0 in range(1, _DO):
            acc = acc + J[di][e0][None, None] * ys[:, e0:e0 + _K]
        accs.append(acc)
    slabs = []
    for dv in range(_D):
        terms = [accs[di][:, dv - di]
                 for di in range(max(0, dv - _K + 1), min(_DO - 1, dv) + 1)]
        s = terms[0]
        for t in terms[1:]:
            s = s + t
        slabs.append(s[:, None])
    folded = jnp.concatenate(slabs, axis=1)

    di2 = lax.broadcasted_iota(jnp.int32, (_D, _P), 0).astype(jnp.float32)
    pi = lax.broadcasted_iota(jnp.int32, (_D, _P), 1)
    hvi = (pi // _H).astype(jnp.float32)
    wvi = (pi % _H).astype(jnp.float32)
    covD = jnp.minimum(jnp.minimum(di2 + 1.0, float(_K)), float(_D) - di2)
    covH = (jnp.minimum(hvi, float(_HO - 1))
            - jnp.maximum(hvi - float(_K - 1), 0.0) + 1.0)
    covW = (jnp.minimum(wvi, float(_HO - 1))
            - jnp.maximum(wvi - float(_K - 1), 0.0) + 1.0)
    wgt = covD * covH * covW

    total = jnp.sum(jnp.abs(xs * wgt[None] - folded))
    out_ref[0, 0] = total / float(_C * _D * _P)


def kernel(x, y):
    xs = x.reshape(_C, _D, _H, _H)
    ys = y.reshape(_C, _D, _H, _H)

    dstack = pl.pallas_call(
        _dist_kernel,
        out_shape=jax.ShapeDtypeStruct((_DO, _NFLAT), jnp.float32),
    )(xs, ys)

    mesh = plsc.VectorSubcoreMesh(core_axis_name="c", subcore_axis_name="s")
    sc_argmin = functools.partial(
        pl.kernel, mesh=mesh,
        out_type=jax.ShapeDtypeStruct((_NFLAT,), jnp.int32),
        scratch_types=[
            pltpu.VMEM((_DO, _CHUNK), jnp.float32),
            pltpu.VMEM((_CHUNK,), jnp.int32),
            pltpu.SemaphoreType.DMA,
        ],
    )(_sc_argmin)
    nns_flat = sc_argmin(dstack)

    out = pl.pallas_call(
        _fold_kernel,
        out_shape=jax.ShapeDtypeStruct((1, 1), jnp.float32),
        out_specs=pl.BlockSpec(memory_space=pltpu.SMEM),
    )(nns_flat, xs, ys)
    return out.reshape(())
